# baseline (speedup 1.0000x reference)
import functools

import jax
import jax.numpy as jnp
from jax import lax
from jax.experimental import pallas as pl
from jax.experimental.pallas import tpu as pltpu
from jax.experimental.pallas import tpu_sc as plsc

NC = 2
NS = 16
NW = NC * NS

POS_ROWS = 64
XB = 16


def _make_sc_kernel(B, S, D, dtype):
    rows_per_w = S // NW
    nsub = rows_per_w // POS_ROWS
    kblocks = POS_ROWS // XB
    mesh = plsc.VectorSubcoreMesh(core_axis_name="c", subcore_axis_name="s")

    @functools.partial(
        pl.kernel,
        mesh=mesh,
        out_type=jax.ShapeDtypeStruct((B, S, D), dtype),
        scratch_types=[
            pltpu.VMEM((POS_ROWS, D), dtype),
            pltpu.VMEM((XB, D), dtype),
            pltpu.VMEM((XB, D), dtype),
            pltpu.SemaphoreType.DMA,
            pltpu.SemaphoreType.DMA,
            pltpu.SemaphoreType.DMA,
            pltpu.SemaphoreType.DMA,
        ],
    )
    def k(x_hbm, pos_hbm, out_hbm, pos_buf, xb0, xb1, ls0, ls1, ss0, ss1):
        xbufs = (xb0, xb1)
        lsems = (ls0, ls1)
        ssems = (ss0, ss1)
        wid = lax.axis_index("s") * NC + lax.axis_index("c")
        s0 = wid * rows_per_w

        def compute_block(xbuf, krow):
            def row_body(r, _):
                pr = krow + r
                for u in range(D // 16):
                    v = pos_buf[pr, pl.ds(u * 16, 16)]
                    plsc.addupdate(xbuf.at[r, pl.ds(u * 16, 16)], v)
                return 0

            lax.fori_loop(0, XB, row_body, 0)

        def sub_body(sub, _):
            sub_row = s0 + sub * POS_ROWS
            pltpu.sync_copy(pos_hbm.at[pl.ds(sub_row, POS_ROWS), :], pos_buf)

            def b_body(b, _):
                def pair_body(i, _):
                    for p in range(2):
                        kk = 2 * i + p
                        rows = sub_row + kk * XB
                        xbuf, lsem, ssem = xbufs[p], lsems[p], ssems[p]

                        def wait_store(xbuf=xbuf, ssem=ssem, rows=rows, b=b):
                            pltpu.make_async_copy(
                                xbuf, out_hbm.at[b, pl.ds(rows, XB), :], ssem
                            ).wait()

                        pl.when(sub + b + i >= 1)(wait_store)
                        pltpu.async_copy(
                            x_hbm.at[b, pl.ds(rows, XB), :], xbuf, lsem
                        ).wait()
                        compute_block(xbuf, kk * XB)
                        pltpu.async_copy(
                            xbuf, out_hbm.at[b, pl.ds(rows, XB), :], ssem
                        )
                    return 0

                lax.fori_loop(0, kblocks // 2, pair_body, 0)
                return 0

            lax.fori_loop(0, B, b_body, 0)
            return 0

        lax.fori_loop(0, nsub, sub_body, 0)

        for p in range(2):
            pltpu.make_async_copy(
                xbufs[p], out_hbm.at[0, pl.ds(0, XB), :], ssems[p]
            ).wait()

    return k


def kernel(x, pos_table):
    B, S, D = x.shape
    pos = pos_table[:S]
    return _make_sc_kernel(B, S, D, x.dtype)(x, pos)


# speedup vs baseline: 1.3157x; 1.3157x over previous
import functools

import jax
import jax.numpy as jnp
from jax import lax
from jax.experimental import pallas as pl
from jax.experimental.pallas import tpu as pltpu
from jax.experimental.pallas import tpu_sc as plsc

NC = 2
NS = 16
NW = NC * NS

POS_ROWS = 32
XB = 16
NBUF = 4


def _make_sc_kernel(B, S, D, dtype):
    rows_per_w = S // NW
    nsub = rows_per_w // POS_ROWS
    kblocks = POS_ROWS // XB
    nblocks = nsub * B * kblocks
    mesh = plsc.VectorSubcoreMesh(core_axis_name="c", subcore_axis_name="s")

    @functools.partial(
        pl.kernel,
        mesh=mesh,
        out_type=jax.ShapeDtypeStruct((B, S, D), dtype),
        scratch_types=[
            pltpu.VMEM((POS_ROWS, D), dtype),
            pltpu.VMEM((XB, D), dtype),
            pltpu.VMEM((XB, D), dtype),
            pltpu.VMEM((XB, D), dtype),
            pltpu.VMEM((XB, D), dtype),
            pltpu.SemaphoreType.DMA,
            pltpu.SemaphoreType.DMA,
            pltpu.SemaphoreType.DMA,
            pltpu.SemaphoreType.DMA,
            pltpu.SemaphoreType.DMA,
            pltpu.SemaphoreType.DMA,
            pltpu.SemaphoreType.DMA,
            pltpu.SemaphoreType.DMA,
        ],
    )
    def k(x_hbm, pos_hbm, out_hbm, pos_buf, xb0, xb1, xb2, xb3,
          ls0, ls1, ls2, ls3, ss0, ss1, ss2, ss3):
        xbufs = (xb0, xb1, xb2, xb3)
        lsems = (ls0, ls1, ls2, ls3)
        ssems = (ss0, ss1, ss2, ss3)
        wid = lax.axis_index("s") * NC + lax.axis_index("c")
        s0 = wid * rows_per_w

        def block_rows(g):
            return s0 + (g >> 3) * POS_ROWS + (g & 1) * XB

        def block_b(g):
            return (g >> 1) & (B - 1)

        def compute_block(xbuf, krow):
            def row_body(r, _):
                pr = krow + r
                for u in range(D // 16):
                    v = pos_buf[pr, pl.ds(u * 16, 16)]
                    plsc.addupdate(xbuf.at[r, pl.ds(u * 16, 16)], v)
                return 0

            lax.fori_loop(0, XB, row_body, 0)

        for g in range(2):
            pltpu.async_copy(
                x_hbm.at[0, pl.ds(s0 + g * XB, XB), :], xbufs[g], lsems[g]
            )

        def outer_body(i, _):
            base = i * NBUF
            for p in range(NBUF):
                g = base + p
                q = (p + 2) % NBUF

                def pos_reload(g=g):
                    pltpu.sync_copy(
                        pos_hbm.at[pl.ds(s0 + (g >> 3) * POS_ROWS, POS_ROWS), :],
                        pos_buf,
                    )

                pl.when((g & (POS_ROWS * B // XB - 1)) == 0)(pos_reload)
                pltpu.make_async_copy(
                    x_hbm.at[0, pl.ds(0, XB), :], xbufs[p], lsems[p]
                ).wait()
                compute_block(xbufs[p], (g & 1) * XB)
                pltpu.async_copy(
                    xbufs[p], out_hbm.at[block_b(g), pl.ds(block_rows(g), XB), :],
                    ssems[p],
                )

                def prefetch(g=g, q=q):
                    g2 = g + 2

                    def wait_store(q=q):
                        pltpu.make_async_copy(
                            xbufs[q], out_hbm.at[0, pl.ds(0, XB), :], ssems[q]
                        ).wait()

                    pl.when(g >= 2)(wait_store)
                    pltpu.async_copy(
                        x_hbm.at[block_b(g2), pl.ds(block_rows(g2), XB), :],
                        xbufs[q], lsems[q],
                    )

                pl.when(g + 2 < nblocks)(prefetch)
            return 0

        lax.fori_loop(0, nblocks // NBUF, outer_body, 0)

        for p in range(NBUF):
            pltpu.make_async_copy(
                xbufs[p], out_hbm.at[0, pl.ds(0, XB), :], ssems[p]
            ).wait()

    return k


def kernel(x, pos_table):
    B, S, D = x.shape
    pos = pos_table[:S]
    return _make_sc_kernel(B, S, D, x.dtype)(x, pos)


# speedup vs baseline: 1.5898x; 1.2083x over previous
import functools

import jax
import jax.numpy as jnp
from jax import lax
from jax.experimental import pallas as pl
from jax.experimental.pallas import tpu as pltpu
from jax.experimental.pallas import tpu_sc as plsc

NC = 2
NS = 16
NW = NC * NS

POS_ROWS = 32
XB = 16
NBUF = 4


def _make_sc_kernel(B, S, D, dtype):
    rows_per_w = S // NW
    nsub = rows_per_w // POS_ROWS
    kblocks = POS_ROWS // XB
    nblocks = nsub * B * kblocks
    mesh = plsc.VectorSubcoreMesh(core_axis_name="c", subcore_axis_name="s")

    @functools.partial(
        pl.kernel,
        mesh=mesh,
        out_type=jax.ShapeDtypeStruct((B, S, D), dtype),
        scratch_types=[
            pltpu.VMEM((POS_ROWS, D), dtype),
            pltpu.VMEM((XB, D), dtype),
            pltpu.VMEM((XB, D), dtype),
            pltpu.VMEM((XB, D), dtype),
            pltpu.VMEM((XB, D), dtype),
            pltpu.SemaphoreType.DMA,
            pltpu.SemaphoreType.DMA,
            pltpu.SemaphoreType.DMA,
            pltpu.SemaphoreType.DMA,
            pltpu.SemaphoreType.DMA,
            pltpu.SemaphoreType.DMA,
            pltpu.SemaphoreType.DMA,
            pltpu.SemaphoreType.DMA,
        ],
    )
    def k(x_hbm, pos_hbm, out_hbm, pos_buf, xb0, xb1, xb2, xb3,
          ls0, ls1, ls2, ls3, ss0, ss1, ss2, ss3):
        xbufs = (xb0, xb1, xb2, xb3)
        lsems = (ls0, ls1, ls2, ls3)
        ssems = (ss0, ss1, ss2, ss3)
        wid = lax.axis_index("s") * NC + lax.axis_index("c")
        s0 = wid * rows_per_w

        def block_rows(g):
            return s0 + (g >> 3) * POS_ROWS + (g & 1) * XB

        def block_b(g):
            return (g >> 1) & (B - 1)

        def compute_block(xbuf, krow):
            @plsc.parallel_loop(0, XB, step=1, unroll=4)
            def row_body(r):
                pr = krow + r
                for u in range(D // 16):
                    v = pos_buf[pr, pl.ds(u * 16, 16)]
                    plsc.addupdate(xbuf.at[r, pl.ds(u * 16, 16)], v)

        for g in range(2):
            pltpu.async_copy(
                x_hbm.at[0, pl.ds(s0 + g * XB, XB), :], xbufs[g], lsems[g]
            )

        def outer_body(i, _):
            base = i * NBUF
            for p in range(NBUF):
                g = base + p
                q = (p + 2) % NBUF

                def pos_reload(g=g):
                    pltpu.sync_copy(
                        pos_hbm.at[pl.ds(s0 + (g >> 3) * POS_ROWS, POS_ROWS), :],
                        pos_buf,
                    )

                pl.when((g & (POS_ROWS * B // XB - 1)) == 0)(pos_reload)
                pltpu.make_async_copy(
                    x_hbm.at[0, pl.ds(0, XB), :], xbufs[p], lsems[p]
                ).wait()
                compute_block(xbufs[p], (g & 1) * XB)
                pltpu.async_copy(
                    xbufs[p], out_hbm.at[block_b(g), pl.ds(block_rows(g), XB), :],
                    ssems[p],
                )

                def prefetch(g=g, q=q):
                    g2 = g + 2

                    def wait_store(q=q):
                        pltpu.make_async_copy(
                            xbufs[q], out_hbm.at[0, pl.ds(0, XB), :], ssems[q]
                        ).wait()

                    pl.when(g >= 2)(wait_store)
                    pltpu.async_copy(
                        x_hbm.at[block_b(g2), pl.ds(block_rows(g2), XB), :],
                        xbufs[q], lsems[q],
                    )

                pl.when(g + 2 < nblocks)(prefetch)
            return 0

        lax.fori_loop(0, nblocks // NBUF, outer_body, 0)

        for p in range(NBUF):
            pltpu.make_async_copy(
                xbufs[p], out_hbm.at[0, pl.ds(0, XB), :], ssems[p]
            ).wait()

    return k


def kernel(x, pos_table):
    B, S, D = x.shape
    pos = pos_table[:S]
    return _make_sc_kernel(B, S, D, x.dtype)(x, pos)


# speedup vs baseline: 2.3147x; 1.4559x over previous
import functools

import jax
import jax.numpy as jnp
from jax import lax
from jax.experimental import pallas as pl
from jax.experimental.pallas import tpu as pltpu
from jax.experimental.pallas import tpu_sc as plsc

NC = 2
NS = 16
NW = NC * NS

XB = 16
NBUF = 4


def _make_sc_kernel(B, S, D, dtype):
    rows_per_w = S // NW
    nsub = rows_per_w // XB
    nblocks = nsub * B
    mesh = plsc.VectorSubcoreMesh(core_axis_name="c", subcore_axis_name="s")

    @functools.partial(
        pl.kernel,
        mesh=mesh,
        out_type=jax.ShapeDtypeStruct((B, S, D), dtype),
        scratch_types=[
            pltpu.VMEM((2 * XB, D), dtype),
            pltpu.VMEM((XB, D), dtype),
            pltpu.VMEM((XB, D), dtype),
            pltpu.VMEM((XB, D), dtype),
            pltpu.VMEM((XB, D), dtype),
            pltpu.SemaphoreType.DMA,
            pltpu.SemaphoreType.DMA,
            pltpu.SemaphoreType.DMA,
            pltpu.SemaphoreType.DMA,
            pltpu.SemaphoreType.DMA,
            pltpu.SemaphoreType.DMA,
            pltpu.SemaphoreType.DMA,
            pltpu.SemaphoreType.DMA,
            pltpu.SemaphoreType.DMA,
        ],
    )
    def k(x_hbm, pos_hbm, out_hbm, pos_buf, xb0, xb1, xb2, xb3,
          psem, ls0, ls1, ls2, ls3, ss0, ss1, ss2, ss3):
        xbufs = (xb0, xb1, xb2, xb3)
        lsems = (ls0, ls1, ls2, ls3)
        ssems = (ss0, ss1, ss2, ss3)
        wid = lax.axis_index("s") * NC + lax.axis_index("c")
        s0 = wid * rows_per_w

        def compute_block(xbuf, h):
            @plsc.parallel_loop(0, XB, step=1, unroll=4)
            def row_body(r):
                pr = h + r
                for u in range(D // 16):
                    v = pos_buf[pr, pl.ds(u * 16, 16)]
                    plsc.addupdate(xbuf.at[r, pl.ds(u * 16, 16)], v)

        def wait_pos():
            pltpu.make_async_copy(
                pos_hbm.at[pl.ds(0, XB), :],
                pos_buf.at[pl.ds(0, XB), :], psem,
            ).wait()

        pltpu.async_copy(
            pos_hbm.at[pl.ds(s0, XB), :], pos_buf.at[pl.ds(0, XB), :], psem
        )
        for g in range(2):
            pltpu.async_copy(
                x_hbm.at[g, pl.ds(s0, XB), :], xbufs[g], lsems[g]
            )

        def sub_body(sub, _):
            h = (sub & 1) * XB
            rows = s0 + sub * XB
            wait_pos()

            def pos_prefetch():
                nxt = sub + 1
                pltpu.async_copy(
                    pos_hbm.at[pl.ds(s0 + nxt * XB, XB), :],
                    pos_buf.at[pl.ds((nxt & 1) * XB, XB), :], psem,
                )

            pl.when(sub + 1 < nsub)(pos_prefetch)

            for p in range(B):
                g = sub * B + p
                q = (p + 2) % NBUF

                pltpu.make_async_copy(
                    x_hbm.at[0, pl.ds(0, XB), :], xbufs[p], lsems[p]
                ).wait()
                compute_block(xbufs[p], h)
                pltpu.async_copy(
                    xbufs[p], out_hbm.at[p, pl.ds(rows, XB), :], ssems[p]
                )

                def prefetch(g=g, q=q):
                    g2 = g + 2
                    b2 = g2 & (B - 1)
                    rows2 = s0 + (g2 >> 2) * XB

                    def wait_store(q=q):
                        pltpu.make_async_copy(
                            xbufs[q], out_hbm.at[0, pl.ds(0, XB), :], ssems[q]
                        ).wait()

                    pl.when(g >= 2)(wait_store)
                    pltpu.async_copy(
                        x_hbm.at[b2, pl.ds(rows2, XB), :], xbufs[q], lsems[q]
                    )

                pl.when(g + 2 < nblocks)(prefetch)
            return 0

        lax.fori_loop(0, nsub, sub_body, 0)

        for p in range(NBUF):
            pltpu.make_async_copy(
                xbufs[p], out_hbm.at[0, pl.ds(0, XB), :], ssems[p]
            ).wait()

    return k


def kernel(x, pos_table):
    B, S, D = x.shape
    pos = pos_table[:S]
    return _make_sc_kernel(B, S, D, x.dtype)(x, pos)


# speedup vs baseline: 2.4376x; 1.0531x over previous
import functools

import jax
import jax.numpy as jnp
from jax import lax
from jax.experimental import pallas as pl
from jax.experimental.pallas import tpu as pltpu
from jax.experimental.pallas import tpu_sc as plsc

NC = 2
NS = 16
NW = NC * NS

XB = 16
NBUF = 4


def _make_sc_kernel(B, S, D, dtype):
    rows_per_w = S // NW
    nsub = rows_per_w // XB
    nblocks = nsub * B
    mesh = plsc.VectorSubcoreMesh(core_axis_name="c", subcore_axis_name="s")

    @functools.partial(
        pl.kernel,
        mesh=mesh,
        out_type=jax.ShapeDtypeStruct((B, S, D), dtype),
        scratch_types=[
            pltpu.VMEM((2 * XB, D), dtype),
            pltpu.VMEM((XB, D), dtype),
            pltpu.VMEM((XB, D), dtype),
            pltpu.VMEM((XB, D), dtype),
            pltpu.VMEM((XB, D), dtype),
            pltpu.SemaphoreType.DMA,
            pltpu.SemaphoreType.DMA,
            pltpu.SemaphoreType.DMA,
            pltpu.SemaphoreType.DMA,
            pltpu.SemaphoreType.DMA,
            pltpu.SemaphoreType.DMA,
            pltpu.SemaphoreType.DMA,
            pltpu.SemaphoreType.DMA,
            pltpu.SemaphoreType.DMA,
        ],
    )
    def k(x_hbm, pos_hbm, out_hbm, pos_buf, xb0, xb1, xb2, xb3,
          psem, ls0, ls1, ls2, ls3, ss0, ss1, ss2, ss3):
        xbufs = (xb0, xb1, xb2, xb3)
        lsems = (ls0, ls1, ls2, ls3)
        ssems = (ss0, ss1, ss2, ss3)
        wid = lax.axis_index("s") * NC + lax.axis_index("c")
        s0 = wid * rows_per_w

        def compute_block(xbuf, h):
            @plsc.parallel_loop(0, XB, step=1, unroll=4)
            def row_body(r):
                pr = h + r
                for u in range(D // 16):
                    v = pos_buf[pr, pl.ds(u * 16, 16)]
                    plsc.addupdate(xbuf.at[r, pl.ds(u * 16, 16)], v)

        def wait_pos():
            pltpu.make_async_copy(
                pos_hbm.at[pl.ds(0, XB), :],
                pos_buf.at[pl.ds(0, XB), :], psem,
            ).wait()

        pltpu.async_copy(
            pos_hbm.at[pl.ds(s0, XB), :], pos_buf.at[pl.ds(0, XB), :], psem
        )
        for g in range(2):
            pltpu.async_copy(
                x_hbm.at[g, pl.ds(s0, XB), :], xbufs[g], lsems[g]
            )

        def sub_body(sub, _):
            h = (sub & 1) * XB
            rows = s0 + sub * XB
            wait_pos()

            def pos_prefetch():
                nxt = sub + 1
                pltpu.async_copy(
                    pos_hbm.at[pl.ds(s0 + nxt * XB, XB), :],
                    pos_buf.at[pl.ds((nxt & 1) * XB, XB), :], psem,
                )

            pl.when(sub + 1 < nsub)(pos_prefetch)

            for p in range(B):
                g = sub * B + p
                q = (p + 2) % NBUF

                pltpu.make_async_copy(
                    x_hbm.at[0, pl.ds(0, XB), :], xbufs[p], lsems[p]
                ).wait()

                def prefetch(g=g, q=q):
                    g2 = g + 2
                    b2 = g2 & (B - 1)
                    rows2 = s0 + (g2 >> 2) * XB

                    def wait_store(q=q):
                        pltpu.make_async_copy(
                            xbufs[q], out_hbm.at[0, pl.ds(0, XB), :], ssems[q]
                        ).wait()

                    pl.when(g >= 2)(wait_store)
                    pltpu.async_copy(
                        x_hbm.at[b2, pl.ds(rows2, XB), :], xbufs[q], lsems[q]
                    )

                pl.when(g + 2 < nblocks)(prefetch)

                compute_block(xbufs[p], h)
                pltpu.async_copy(
                    xbufs[p], out_hbm.at[p, pl.ds(rows, XB), :], ssems[p]
                )
            return 0

        lax.fori_loop(0, nsub, sub_body, 0)

        for p in range(NBUF):
            pltpu.make_async_copy(
                xbufs[p], out_hbm.at[0, pl.ds(0, XB), :], ssems[p]
            ).wait()

    return k


def kernel(x, pos_table):
    B, S, D = x.shape
    pos = pos_table[:S]
    return _make_sc_kernel(B, S, D, x.dtype)(x, pos)
